# baseline (speedup 1.0000x reference)
import functools

import jax
import jax.numpy as jnp
from jax import lax
from jax.experimental import pallas as pl
from jax.experimental.pallas import tpu as pltpu
from jax.experimental.pallas import tpu_sc as plsc

_B = 16384
_L = 16
_NC = 2
_NS = 16
_NW = _NC * _NS
_CHUNK = _B // _NW
_NITER = _CHUNK // _L

_mesh = plsc.VectorSubcoreMesh(core_axis_name="c", subcore_axis_name="s")


@functools.partial(
    pl.kernel,
    mesh=_mesh,
    out_type=(
        jax.ShapeDtypeStruct((_NC, _L), jnp.float32),
        jax.ShapeDtypeStruct((_NC, _L), jnp.float32),
    ),
    scratch_types=[
        pltpu.VMEM((_CHUNK,), jnp.float32),
        pltpu.VMEM((_CHUNK,), jnp.float32),
        pltpu.VMEM((_CHUNK,), jnp.float32),
        pltpu.VMEM((_L,), jnp.float32),
        pltpu.VMEM_SHARED((_NS * _L,), jnp.float32),
        pltpu.VMEM_SHARED((_NS * _L,), jnp.float32),
        pltpu.VMEM((_NS * _L,), jnp.float32),
        pltpu.VMEM((_NS * _L,), jnp.float32),
    ],
)
def _masked_mse_sc(o_hbm, t_hbm, m_hbm, sse_out, cnt_out,
                   o_v, t_v, m_v, stage_v, sse_sh, cnt_sh, sse_rd, cnt_rd):
    cid = lax.axis_index("c")
    sid = lax.axis_index("s")
    base = (cid * _NS + sid) * _CHUNK

    pltpu.sync_copy(o_hbm.at[pl.ds(base, _CHUNK)], o_v)
    pltpu.sync_copy(t_hbm.at[pl.ds(base, _CHUNK)], t_v)
    pltpu.sync_copy(m_hbm.at[pl.ds(base, _CHUNK)], m_v)

    def body(i, carry):
        acc_s, acc_c = carry
        o = o_v[pl.ds(i * _L, _L)]
        t = t_v[pl.ds(i * _L, _L)]
        m = m_v[pl.ds(i * _L, _L)]
        d = o - t
        return acc_s + d * d * m, acc_c + m

    zero = jnp.zeros((_L,), jnp.float32)
    acc_s, acc_c = lax.fori_loop(0, _NITER, body, (zero, zero))

    stage_v[...] = acc_s
    pltpu.sync_copy(stage_v, sse_sh.at[pl.ds(sid * _L, _L)])
    stage_v[...] = acc_c
    pltpu.sync_copy(stage_v, cnt_sh.at[pl.ds(sid * _L, _L)])
    plsc.subcore_barrier()

    @pl.when(sid == 0)
    def _finalize():
        pltpu.sync_copy(sse_sh, sse_rd)
        pltpu.sync_copy(cnt_sh, cnt_rd)

        def body2(i, carry):
            a_s, a_c = carry
            return (a_s + sse_rd[pl.ds(i * _L, _L)],
                    a_c + cnt_rd[pl.ds(i * _L, _L)])

        tot_s, tot_c = lax.fori_loop(0, _NS, body2, (zero, zero))

        sse = tot_s[0]
        cnt = tot_c[0]
        for i in range(1, _L):
            sse = sse + tot_s[i]
            cnt = cnt + tot_c[i]
        stage_v[...] = jnp.broadcast_to(sse, (_L,))
        pltpu.sync_copy(stage_v, sse_out.at[cid])
        stage_v[...] = jnp.broadcast_to(cnt, (_L,))
        pltpu.sync_copy(stage_v, cnt_out.at[cid])


def kernel(outputs, targets, precondition):
    o = outputs.reshape(_B)
    t = targets.reshape(_B)
    m = precondition.reshape(_B).astype(jnp.float32)
    sse_p, cnt_p = _masked_mse_sc(o, t, m)
    sse = sse_p[0, 0] + sse_p[1, 0]
    cnt = cnt_p[0, 0] + cnt_p[1, 0]
    return jnp.where(cnt > 0.0, sse / jnp.maximum(cnt, 1.0),
                     jnp.float32(0.0))


# speedup vs baseline: 1.3695x; 1.3695x over previous
import functools

import jax
import jax.numpy as jnp
from jax import lax
from jax.experimental import pallas as pl
from jax.experimental.pallas import tpu as pltpu
from jax.experimental.pallas import tpu_sc as plsc

_B = 16384
_L = 16
_NS = 16
_CHUNK = _B // _NS
_NITER = _CHUNK // _L

_mesh = plsc.VectorSubcoreMesh(
    core_axis_name="c", subcore_axis_name="s", num_cores=1)


@functools.partial(
    pl.kernel,
    mesh=_mesh,
    out_type=jax.ShapeDtypeStruct((_L,), jnp.float32),
    scratch_types=[
        pltpu.VMEM((_CHUNK,), jnp.float32),
        pltpu.VMEM((_CHUNK,), jnp.float32),
        pltpu.VMEM((_CHUNK,), jnp.float32),
        pltpu.VMEM((_L,), jnp.float32),
        pltpu.VMEM_SHARED((2 * _NS * _L,), jnp.float32),
        pltpu.VMEM((2 * _NS * _L,), jnp.float32),
        pltpu.SemaphoreType.DMA,
    ],
)
def _masked_mse_sc(o_hbm, t_hbm, m_hbm, out_hbm,
                   o_v, t_v, m_v, stage_v, part_sh, part_rd, sem):
    sid = lax.axis_index("s")
    base = sid * _CHUNK

    c1 = pltpu.async_copy(o_hbm.at[pl.ds(base, _CHUNK)], o_v, sem)
    c2 = pltpu.async_copy(t_hbm.at[pl.ds(base, _CHUNK)], t_v, sem)
    c3 = pltpu.async_copy(m_hbm.at[pl.ds(base, _CHUNK)], m_v, sem)
    c1.wait()
    c2.wait()
    c3.wait()

    def body(i, carry):
        acc_s, acc_c = carry
        o = o_v[pl.ds(i * _L, _L)]
        t = t_v[pl.ds(i * _L, _L)]
        m = m_v[pl.ds(i * _L, _L)]
        d = o - t
        return acc_s + d * d * m, acc_c + m

    zero = jnp.zeros((_L,), jnp.float32)
    acc_s, acc_c = lax.fori_loop(0, _NITER, body, (zero, zero))

    stage_v[...] = acc_s
    pltpu.sync_copy(stage_v, part_sh.at[pl.ds(sid * _L, _L)])
    stage_v[...] = acc_c
    pltpu.sync_copy(stage_v, part_sh.at[pl.ds((_NS + sid) * _L, _L)])
    plsc.subcore_barrier()

    @pl.when(sid == 0)
    def _finalize():
        pltpu.sync_copy(part_sh, part_rd)

        def body2(i, carry):
            a_s, a_c = carry
            return (a_s + part_rd[pl.ds(i * _L, _L)],
                    a_c + part_rd[pl.ds((_NS + i) * _L, _L)])

        tot_s, tot_c = lax.fori_loop(0, _NS, body2, (zero, zero))

        sse = tot_s[0]
        cnt = tot_c[0]
        for i in range(1, _L):
            sse = sse + tot_s[i]
            cnt = cnt + tot_c[i]

        one_v = jnp.ones((_L,), jnp.float32)
        cnt_v = jnp.broadcast_to(cnt, (_L,))
        cnt_c = jnp.maximum(cnt_v, one_v)
        inv = one_v / cnt_c
        inv = inv * (2.0 - cnt_c * inv)
        inv = inv * (2.0 - cnt_c * inv)
        mse_v = jnp.where(cnt_v > 0.0,
                          jnp.broadcast_to(sse, (_L,)) * inv,
                          jnp.zeros((_L,), jnp.float32))
        stage_v[...] = mse_v
        pltpu.sync_copy(stage_v, out_hbm)


def kernel(outputs, targets, precondition):
    o = outputs.reshape(_B)
    t = targets.reshape(_B)
    m = precondition.reshape(_B).astype(jnp.float32)
    out = _masked_mse_sc(o, t, m)
    return out[0]
